# baseline (speedup 1.0000x reference)
import functools

import jax
import jax.numpy as jnp
from jax import lax
from jax.experimental import pallas as pl
from jax.experimental.pallas import tpu as pltpu
from jax.experimental.pallas import tpu_sc as plsc


@functools.lru_cache(maxsize=None)
def _build(B, T, V, D):
    info = plsc.get_sparse_core_info()
    NC, NS = info.num_cores, info.num_subcores
    NW = NC * NS
    BL = B // NW
    assert BL == 128 and D == 64 and T % 2 == 0
    mesh = plsc.VectorSubcoreMesh(core_axis_name="c", subcore_axis_name="s")

    @functools.partial(
        pl.kernel,
        mesh=mesh,
        out_type=jax.ShapeDtypeStruct((T, D, B), jnp.float32),
        scratch_types=[
            pltpu.VMEM((T, BL), jnp.int32),
            pltpu.VMEM((T, 2 * D), jnp.float32),
            pltpu.VMEM((BL, 2 * D), jnp.float32),
            pltpu.VMEM((BL, 2 * D), jnp.float32),
            pltpu.VMEM((D, BL), jnp.float32),
            pltpu.VMEM((D, BL), jnp.float32),
            pltpu.SemaphoreType.DMA,
            pltpu.SemaphoreType.DMA,
            pltpu.SemaphoreType.DMA,
            pltpu.SemaphoreType.DMA,
        ],
        compiler_params=pltpu.CompilerParams(needs_layout_passes=False),
    )
    def k(xT_hbm, tok2_hbm, pos2_hbm, out_hbm,
          xw, posb, rows0, rows1, ob0, ob1, g0, g1, s0, s1):
        wid = lax.axis_index("s") * NC + lax.axis_index("c")
        b0 = wid * BL
        pltpu.sync_copy(xT_hbm.at[:, pl.ds(b0, BL)], xw)
        pltpu.sync_copy(pos2_hbm, posb)

        didx = [lax.iota(jnp.int32, 16) + (c * 16) for c in range(4)]
        pltpu.async_copy(tok2_hbm.at[xw.at[0]], rows0, g0)
        pltpu.async_copy(tok2_hbm.at[xw.at[1]], rows1, g1)

        def phase(t, i, rows, ob, g, s):
            pltpu.make_async_copy(tok2_hbm.at[xw.at[0]], rows, g).wait()

            @pl.when(i > 0)
            def _():
                pltpu.make_async_copy(ob, out_hbm.at[0, :, pl.ds(b0, BL)], s).wait()

            pvs = [posb[t, pl.ds(c * 16, 16)] for c in range(4)]

            def body(j, carry):
                for u in range(2):
                    b = 2 * j + u
                    bs = jnp.full((16,), b, jnp.int32)
                    for c in range(4):
                        v = rows[b, pl.ds(c * 16, 16)] + pvs[c]
                        plsc.store_scatter(ob, [didx[c], bs], v)
                return carry

            lax.fori_loop(0, BL // 2, body, 0)
            pltpu.async_copy(ob, out_hbm.at[t, :, pl.ds(b0, BL)], s)

            @pl.when(t + 2 < T)
            def _():
                pltpu.async_copy(tok2_hbm.at[xw.at[t + 2]], rows, g)

        def pair_body(i, carry):
            phase(2 * i, i, rows0, ob0, g0, s0)
            phase(2 * i + 1, i, rows1, ob1, g1, s1)
            return carry

        lax.fori_loop(0, T // 2, pair_body, 0)
        pltpu.make_async_copy(ob0, out_hbm.at[0, :, pl.ds(b0, BL)], s0).wait()
        pltpu.make_async_copy(ob1, out_hbm.at[0, :, pl.ds(b0, BL)], s1).wait()

    return k


def kernel(x, token_table, pos_table):
    B, T = x.shape
    V, D = token_table.shape
    xT = x.astype(jnp.int32).T
    tok2 = jnp.concatenate([token_table, token_table], axis=1)
    pos2 = jnp.concatenate([pos_table, pos_table], axis=1)
    out = _build(B, T, V, D)(xT, tok2, pos2)
    return jnp.transpose(out, (2, 0, 1))


# speedup vs baseline: 1.7038x; 1.7038x over previous
import functools

import jax
import jax.numpy as jnp
from jax import lax
from jax.experimental import pallas as pl
from jax.experimental.pallas import tpu as pltpu
from jax.experimental.pallas import tpu_sc as plsc


@functools.lru_cache(maxsize=None)
def _build(B, T, V, D):
    info = plsc.get_sparse_core_info()
    NC, NS = info.num_cores, info.num_subcores
    NW = NC * NS
    assert B % NW == 0
    bpw = B // NW
    CB = 2
    assert bpw % CB == 0
    nchunk = bpw // CB
    assert nchunk % 3 == 1 and D % 16 == 0
    mesh = plsc.VectorSubcoreMesh(core_axis_name="c", subcore_axis_name="s")

    @functools.partial(
        pl.kernel,
        mesh=mesh,
        out_type=jax.ShapeDtypeStruct((B, T, D), jnp.float32),
        scratch_types=[
            pltpu.VMEM((bpw, T), jnp.int32),
            pltpu.VMEM((T, D), jnp.float32),
            pltpu.VMEM((CB, T, D), jnp.float32),
            pltpu.VMEM((CB, T, D), jnp.float32),
            pltpu.VMEM((CB, T, D), jnp.float32),
            pltpu.SemaphoreType.DMA,
            pltpu.SemaphoreType.DMA,
            pltpu.SemaphoreType.DMA,
            pltpu.SemaphoreType.DMA,
            pltpu.SemaphoreType.DMA,
            pltpu.SemaphoreType.DMA,
        ],
        compiler_params=pltpu.CompilerParams(use_tc_tiling_on_sc=False),
    )
    def k(x_hbm, tok_hbm, pos_hbm, out_hbm,
          xw, pos_v, r0, r1, r2, g0, g1, g2, s0, s1, s2):
        rows, gs, ss = [r0, r1, r2], [g0, g1, g2], [s0, s1, s2]
        wid = lax.axis_index("s") * NC + lax.axis_index("c")
        base_b = wid * bpw
        pltpu.sync_copy(x_hbm.at[pl.ds(base_b, bpw)], xw)
        pltpu.sync_copy(pos_hbm, pos_v)

        def issue_gather(g, p):
            for j in range(CB):
                pltpu.async_copy(tok_hbm.at[xw.at[g * CB + j]],
                                 rows[p].at[j], gs[p])

        def process(g, p):
            cp = pltpu.make_async_copy(tok_hbm.at[xw.at[0]],
                                       rows[p].at[0], gs[p])
            for _ in range(CB):
                cp.wait()

            def tbody(t, carry):
                pv = [pos_v[t, pl.ds(16 * c, 16)] for c in range(D // 16)]
                for j in range(CB):
                    for c in range(D // 16):
                        plsc.addupdate(rows[p].at[j, t, pl.ds(16 * c, 16)],
                                       pv[c])
                return carry

            lax.fori_loop(0, T, tbody, 0)
            pltpu.async_copy(rows[p], out_hbm.at[pl.ds(base_b + g * CB, CB)],
                             ss[p])
            q = (p + 2) % 3

            @pl.when(g + 2 < nchunk)
            def _():
                @pl.when(g > 0)
                def _():
                    pltpu.make_async_copy(
                        rows[q], out_hbm.at[pl.ds(base_b, CB)], ss[q]).wait()
                issue_gather(g + 2, q)

        issue_gather(0, 0)
        issue_gather(1, 1)

        def body3(i, carry):
            for u in range(3):
                process(3 * i + u, u)
            return carry

        lax.fori_loop(0, nchunk // 3, body3, 0)
        process(nchunk - 1, 0)
        for p in (1, 2, 0):
            pltpu.make_async_copy(
                rows[p], out_hbm.at[pl.ds(base_b, CB)], ss[p]).wait()

    return k


def kernel(x, token_table, pos_table):
    B, T = x.shape
    V, D = token_table.shape
    return _build(B, T, V, D)(x.astype(jnp.int32), token_table, pos_table)


# speedup vs baseline: 1.7041x; 1.0002x over previous
import functools

import jax
import jax.numpy as jnp
from jax import lax
from jax.experimental import pallas as pl
from jax.experimental.pallas import tpu as pltpu
from jax.experimental.pallas import tpu_sc as plsc


@functools.lru_cache(maxsize=None)
def _build(B, T, V, D):
    info = plsc.get_sparse_core_info()
    NC, NS = info.num_cores, info.num_subcores
    NW = NC * NS
    assert B % NW == 0
    bpw = B // NW
    CB = 2
    assert bpw % CB == 0
    nchunk = bpw // CB
    assert nchunk % 3 == 1 and D % 16 == 0
    CR = CB * T
    mesh = plsc.VectorSubcoreMesh(core_axis_name="c", subcore_axis_name="s")

    @functools.partial(
        pl.kernel,
        mesh=mesh,
        out_type=jax.ShapeDtypeStruct((B * T, D), jnp.float32),
        scratch_types=[
            pltpu.VMEM((bpw * T,), jnp.int32),
            pltpu.VMEM((T, D), jnp.float32),
            pltpu.VMEM((CR, D), jnp.float32),
            pltpu.VMEM((CR, D), jnp.float32),
            pltpu.VMEM((CR, D), jnp.float32),
            pltpu.SemaphoreType.DMA,
            pltpu.SemaphoreType.DMA,
            pltpu.SemaphoreType.DMA,
            pltpu.SemaphoreType.DMA,
            pltpu.SemaphoreType.DMA,
            pltpu.SemaphoreType.DMA,
        ],
        compiler_params=pltpu.CompilerParams(use_tc_tiling_on_sc=False),
    )
    def k(x_hbm, tok_hbm, pos_hbm, out_hbm,
          xw, pos_v, r0, r1, r2, g0, g1, g2, s0, s1, s2):
        rows, gs, ss = [r0, r1, r2], [g0, g1, g2], [s0, s1, s2]
        wid = lax.axis_index("s") * NC + lax.axis_index("c")
        base_r = wid * (bpw * T)
        pltpu.sync_copy(x_hbm.at[pl.ds(base_r, bpw * T)], xw)
        pltpu.sync_copy(pos_hbm, pos_v)

        def issue_gather(g, p):
            pltpu.async_copy(tok_hbm.at[xw.at[pl.ds(g * CR, CR)]],
                             rows[p], gs[p])

        def process(g, p):
            pltpu.make_async_copy(tok_hbm.at[xw.at[pl.ds(0, CR)]],
                                  rows[p], gs[p]).wait()

            def tbody(t, carry):
                pv = [pos_v[t, pl.ds(16 * c, 16)] for c in range(D // 16)]
                for j in range(CB):
                    for c in range(D // 16):
                        plsc.addupdate(
                            rows[p].at[j * T + t, pl.ds(16 * c, 16)], pv[c])
                return carry

            lax.fori_loop(0, T, tbody, 0)
            pltpu.async_copy(rows[p], out_hbm.at[pl.ds(base_r + g * CR, CR)],
                             ss[p])
            q = (p + 2) % 3

            @pl.when(g + 2 < nchunk)
            def _():
                @pl.when(g > 0)
                def _():
                    pltpu.make_async_copy(
                        rows[q], out_hbm.at[pl.ds(base_r, CR)], ss[q]).wait()
                issue_gather(g + 2, q)

        issue_gather(0, 0)
        issue_gather(1, 1)

        def body3(i, carry):
            for u in range(3):
                process(3 * i + u, u)
            return carry

        lax.fori_loop(0, nchunk // 3, body3, 0)
        process(nchunk - 1, 0)
        for p in (1, 2, 0):
            pltpu.make_async_copy(
                rows[p], out_hbm.at[pl.ds(base_r, CR)], ss[p]).wait()

    return k


def kernel(x, token_table, pos_table):
    B, T = x.shape
    V, D = token_table.shape
    xf = x.astype(jnp.int32).reshape(B * T)
    out = _build(B, T, V, D)(xf, token_table, pos_table)
    return out.reshape(B, T, D)
